# baseline (speedup 1.0000x reference)
import functools

import jax
import jax.numpy as jnp
from jax import lax
from jax.experimental import pallas as pl
from jax.experimental.pallas import tpu as pltpu
from jax.experimental.pallas import tpu_sc as plsc

D = 192
DD = D * D
N = D * DD
ROWS = N // 128

BR = 512

TILES = 16
PER_TILE = N // TILES
CHUNK_E = 6144
NCHUNK = PER_TILE // CHUNK_E
NP = 6
FILL_IT = CHUNK_E // 16

NSEC = 4
QN = N // NSEC
QSH = QN + 128
QPT = QN // TILES


def _tc_idx_body(qp_ref, x_ref, idx_ref, *, s0):
    w = qp_ref[0]
    qx = qp_ref[1]
    qy = qp_ref[2]
    qz = qp_ref[3]

    r0 = pl.program_id(0) * BR
    lane = lax.broadcasted_iota(jnp.int32, (BR, 128), 1)
    f = (r0 + lax.broadcasted_iota(jnp.int32, (BR, 128), 0)) * 128 + lane
    ci = lax.shift_right_logical(lax.shift_right_logical(f, 12) * 58255, 19)
    rem = f - ci * DD
    cj = lax.shift_right_logical(lax.shift_right_logical(rem, 6) * 43691, 17)
    ck = rem - cj * D

    cx = ci.astype(jnp.float32)
    cy = cj.astype(jnp.float32)
    cz = ck.astype(jnp.float32)

    cqw = ((0.0 - qx * cx) - qy * cy) - qz * cz
    cqx = (w * cx + qy * cz) - qz * cy
    cqy = (w * cy + qz * cx) - qx * cz
    cqz = (w * cz + qx * cy) - qy * cx
    nqx = -qx
    nqy = -qy
    nqz = -qz
    rqx = (cqw * nqx + cqx * w + cqy * nqz) - cqz * nqy
    rqy = (cqw * nqy + cqy * w + cqz * nqx) - cqx * nqz
    rqz = (cqw * nqz + cqz * w + cqx * nqy) - cqy * nqx

    ri0 = rqx.astype(jnp.int32)
    ri1 = rqy.astype(jnp.int32)
    ri2 = rqz.astype(jnp.int32)
    valid = ((ri0 >= 0) & (ri0 < D) & (ri1 >= 0) & (ri1 < D)
             & (ri2 >= 0) & (ri2 < D) & (x_ref[...] != 0.0))
    t = ri0 * DD + ri1 * D + ri2
    dummy = QN + lane
    for si in range(2):
        base = (s0 + si) * QN
        ts = t - base
        live = valid & (ts >= 0) & (ts < QN)
        idx_ref[si] = jnp.where(live, ts, dummy)


def _tc_idx(qp, x2d, s0):
    return pl.pallas_call(
        functools.partial(_tc_idx_body, s0=s0),
        grid=(ROWS // BR,),
        in_specs=[
            pl.BlockSpec(memory_space=pltpu.SMEM),
            pl.BlockSpec((BR, 128), lambda i: (i, 0)),
        ],
        out_specs=pl.BlockSpec((2, BR, 128), lambda i: (0, i, 0)),
        out_shape=jax.ShapeDtypeStruct((2, ROWS, 128), jnp.int32),
        compiler_params=pltpu.CompilerParams(
            dimension_semantics=("arbitrary",)),
    )(qp, x2d)


_SC_MESH = plsc.VectorSubcoreMesh(
    core_axis_name="c", subcore_axis_name="s", num_cores=2)


@functools.partial(
    pl.kernel,
    out_type=jax.ShapeDtypeStruct((2 * QN,), jnp.float32),
    mesh=_SC_MESH,
    scratch_types=[
        pltpu.VMEM((CHUNK_E,), jnp.int32),
        pltpu.VMEM((CHUNK_E,), jnp.int32),
        pltpu.VMEM((CHUNK_E,), jnp.float32),
        pltpu.VMEM_SHARED((QSH,), jnp.float32),
        pltpu.SemaphoreType.DMA,
        pltpu.SemaphoreType.DMA,
    ],
)
def _sc_scatter(idx_hbm, out_hbm, ib0, ib1, fbuf, shared, sem0, sem1):
    cid = lax.axis_index("c")
    sid = lax.axis_index("s")

    def _fill(val):
        def body(q, carry):
            fbuf[pl.ds(q * 16, 16)] = jnp.full((16,), val, jnp.float32)
            return carry
        lax.fori_loop(0, FILL_IT, body, 0)

    tilebase = sid * PER_TILE
    outbase = cid * QN
    idxbase = cid * N + tilebase

    _fill(0.0)
    for z in range(QPT // CHUNK_E):
        pltpu.sync_copy(
            fbuf, shared.at[pl.ds(sid * QPT + z * CHUNK_E, CHUNK_E)])
    plsc.subcore_barrier()

    if True:

        _fill(1.0)
        bufs = (ib0, ib1)
        sems = (sem0, sem1)

        def gbody(g, carry):
            gb = idxbase + g * (NP * CHUNK_E)
            pltpu.sync_copy(idx_hbm.at[pl.ds(gb, CHUNK_E)], bufs[0])
            hs = [None,
                  pltpu.async_copy(
                      idx_hbm.at[pl.ds(gb + CHUNK_E, CHUNK_E)],
                      bufs[1], sems[1])]
            for u in range(NP):
                b = u % 2
                if hs[b] is not None:
                    hs[b].wait()
                    hs[b] = None
                pltpu.sync_copy(fbuf, shared.at[bufs[b]])
                if u + 2 < NP:
                    hs[b] = pltpu.async_copy(
                        idx_hbm.at[pl.ds(gb + (u + 2) * CHUNK_E, CHUNK_E)],
                        bufs[b], sems[b])
            return carry
        lax.fori_loop(0, NCHUNK // NP, gbody, 0)
        plsc.subcore_barrier()

        for z in range(QPT // CHUNK_E):
            off = sid * QPT + z * CHUNK_E
            pltpu.sync_copy(shared.at[pl.ds(off, CHUNK_E)],
                            out_hbm.at[pl.ds(outbase + off, CHUNK_E)])
        plsc.subcore_barrier()


@jax.jit
def kernel(x, axis, theta):
    sin_half = jnp.sin(theta / 2.0)
    cos_half = jnp.cos(theta / 2.0)
    qp = jnp.concatenate(
        (jnp.reshape(cos_half, (1,)), axis * sin_half)).astype(jnp.float32)
    x2d = x.reshape(ROWS, 128)
    idx0 = _tc_idx(qp, x2d, 0)
    idx1 = _tc_idx(qp, x2d, 2)
    out0 = _sc_scatter(idx0.reshape(2 * N))
    out1 = _sc_scatter(idx1.reshape(2 * N))
    return jnp.concatenate((out0, out1)).reshape(D, D, D)
